# Initial kernel scaffold; baseline (speedup 1.0000x reference)
import jax
import jax.numpy as jnp
from jax.experimental import pallas as pl


def kernel(x, edge_indices, ego_mask, W1, b1, W2, b2, W_ih, W_hh, b_ih, b_hh, W_fc, b_fc):
    raise NotImplementedError("write your pallas kernel here")


# speedup vs baseline: 11.7718x; 11.7718x over previous
import functools

import jax
import jax.numpy as jnp
from jax import lax
from jax.experimental import pallas as pl
from jax.experimental.pallas import tpu as pltpu
from jax.experimental.pallas import tpu_sc as plsc

N = 10000
T = 20
E = 320000
D_IN = 128
H = 64
D_OUT = 128
B = 25

NC = 2
NS = 16
ROWS_PER_TILE = 640
NPAD = NS * ROWS_PER_TILE
EDGES_PER_TILE = E // NS
CHUNK = 80
NCHUNKS = EDGES_PER_TILE // CHUNK
T_PER_SC = T // NC

@functools.cache
def _make_deg_kernel():
    return pl.kernel(
        _deg_body,
        out_type=jax.ShapeDtypeStruct((T * NPAD,), jnp.float32),
        mesh=plsc.VectorSubcoreMesh(core_axis_name="c", subcore_axis_name="s"),
        compiler_params=pltpu.CompilerParams(use_tc_tiling_on_sc=False),
        scratch_types=[
            pltpu.VMEM((ROWS_PER_TILE,), jnp.float32),
            pltpu.VMEM((CHUNK,), jnp.int32),
            pltpu.VMEM_SHARED((NPAD,), jnp.float32),
            pltpu.SemaphoreType.DMA,
        ],
    )


def _deg_body(dst_hbm, deg_hbm, ones_v, idx_v, acc, sem):
    c = lax.axis_index("c")
    s = lax.axis_index("s")

    def fill_ones(i, _):
        ones_v[pl.ds(i * 16, 16)] = jnp.full((16,), 1.0, jnp.float32)
        return 0

    lax.fori_loop(0, ROWS_PER_TILE // 16, fill_ones, 0)

    my_rows = pl.ds(s * ROWS_PER_TILE, ROWS_PER_TILE)
    for j in range(T_PER_SC):
        t = j * NC + c
        pltpu.sync_copy(ones_v, acc.at[my_rows])
        plsc.subcore_barrier()

        base = t * E + s * EDGES_PER_TILE

        def edge_chunk(k, _):
            off = base + k * CHUNK
            pltpu.sync_copy(dst_hbm.at[pl.ds(off, CHUNK)], idx_v)
            pltpu.sync_copy(ones_v.at[pl.ds(0, CHUNK)], acc.at[idx_v], add=True)
            return 0

        lax.fori_loop(0, NCHUNKS, edge_chunk, 0)
        plsc.subcore_barrier()
        pltpu.sync_copy(acc.at[my_rows],
                        deg_hbm.at[pl.ds(t * NPAD + s * ROWS_PER_TILE,
                                         ROWS_PER_TILE)])


@functools.cache
def _make_agg_kernel():
    return pl.kernel(
        _agg_body,
        out_type=jax.ShapeDtypeStruct((T * NPAD, H), jnp.float32),
        mesh=plsc.VectorSubcoreMesh(core_axis_name="c", subcore_axis_name="s"),
        compiler_params=pltpu.CompilerParams(use_tc_tiling_on_sc=False),
        scratch_types=[
            pltpu.VMEM((CHUNK,), jnp.int32),
            pltpu.VMEM((CHUNK,), jnp.int32),
            pltpu.VMEM((CHUNK, H), jnp.float32),
            pltpu.VMEM((CHUNK, H), jnp.float32),
            pltpu.VMEM_SHARED((NPAD, H), jnp.float32),
            pltpu.SemaphoreType.DMA,
        ],
    )


def _agg_body(y_hbm, src_hbm, dst_hbm, s_hbm, idx_s, idx_d, rows_v, zeros_v,
              acc, sem):
    c = lax.axis_index("c")
    s = lax.axis_index("s")

    def fill_zeros(i, _):
        zeros_v[i // 4, pl.ds((i % 4) * 16, 16)] = jnp.zeros((16,), jnp.float32)
        return 0

    lax.fori_loop(0, CHUNK * (H // 16), fill_zeros, 0)

    for j in range(T_PER_SC):
        t = j * NC + c
        for q in range(ROWS_PER_TILE // CHUNK):
            pltpu.sync_copy(
                zeros_v, acc.at[pl.ds(s * ROWS_PER_TILE + q * CHUNK, CHUNK)])
        plsc.subcore_barrier()

        base = t * E + s * EDGES_PER_TILE
        row_off = t * N

        def edge_chunk(k, _):
            off = base + k * CHUNK
            pltpu.sync_copy(src_hbm.at[pl.ds(off, CHUNK)], idx_s)
            pltpu.sync_copy(dst_hbm.at[pl.ds(off, CHUNK)], idx_d)

            def adj(i, _):
                idx_s[pl.ds(i * 16, 16)] = idx_s[pl.ds(i * 16, 16)] + row_off
                return 0

            lax.fori_loop(0, CHUNK // 16, adj, 0)
            pltpu.async_copy(y_hbm.at[idx_s], rows_v, sem).wait()
            pltpu.sync_copy(rows_v, acc.at[idx_d], add=True)
            return 0

        lax.fori_loop(0, NCHUNKS, edge_chunk, 0)
        plsc.subcore_barrier()
        pltpu.sync_copy(
            acc.at[pl.ds(s * ROWS_PER_TILE, ROWS_PER_TILE)],
            s_hbm.at[pl.ds(t * NPAD + s * ROWS_PER_TILE, ROWS_PER_TILE)])


BN = 2000
NB = N // BN
BN2 = 400


def _mm1_body(x_ref, w_ref, deg_ref, y_ref):
    dinv = lax.rsqrt(deg_ref[0])
    y_ref[0] = (
        jnp.dot(x_ref[0], w_ref[...], preferred_element_type=jnp.float32)
        * dinv)


def _mm1(x, w1, deg3):
    return pl.pallas_call(
        _mm1_body,
        grid=(T, NB),
        in_specs=[
            pl.BlockSpec((1, BN, D_IN), lambda t, i: (t, i, 0)),
            pl.BlockSpec((D_IN, H), lambda t, i: (0, 0)),
            pl.BlockSpec((1, BN, 1), lambda t, i: (t, i, 0)),
        ],
        out_specs=pl.BlockSpec((1, BN, H), lambda t, i: (t, i, 0)),
        out_shape=jax.ShapeDtypeStruct((T, N, H), jnp.float32),
    )(x, w1, deg3)


def _mm2_body(s_ref, y_ref, deg_ref, b_ref, w_ref, out_ref):
    dinv = lax.rsqrt(deg_ref[0])
    z = jnp.maximum(dinv * (s_ref[0] + y_ref[0]) + b_ref[...], 0.0)
    out_ref[0] = (
        jnp.dot(z, w_ref[...], preferred_element_type=jnp.float32) * dinv)


def _mm2(s1, y1, deg3, b1r, w2):
    return pl.pallas_call(
        _mm2_body,
        grid=(T, NB),
        in_specs=[
            pl.BlockSpec((1, BN, H), lambda t, i: (t, i, 0)),
            pl.BlockSpec((1, BN, H), lambda t, i: (t, i, 0)),
            pl.BlockSpec((1, BN, 1), lambda t, i: (t, i, 0)),
            pl.BlockSpec((1, H), lambda t, i: (0, 0)),
            pl.BlockSpec((H, H), lambda t, i: (0, 0)),
        ],
        out_specs=pl.BlockSpec((1, BN, H), lambda t, i: (t, i, 0)),
        out_shape=jax.ShapeDtypeStruct((T, N, H), jnp.float32),
    )(s1, y1, deg3, b1r, w2)


def _sigmoid(x):
    return 1.0 / (1.0 + jnp.exp(-x))


def _final_body(s_ref, y_ref, deg_ref, m_ref, b2_ref, wih_ref, whh_ref,
                bih_ref, bhh_ref, wfc_ref, bfc_ref, out_ref):
    h = jnp.zeros((BN2, H), jnp.float32)
    c = jnp.zeros((BN2, H), jnp.float32)
    bg = bih_ref[...] + bhh_ref[...]
    dn = (((1,), (1,)), ((), ()))
    for t in range(T):
        dinv = lax.rsqrt(deg_ref[t])
        keep = 1.0 - m_ref[t]
        xo = (dinv * (s_ref[t] + y_ref[t]) + b2_ref[...]) * keep
        g = [
            lax.dot_general(xo, wih_ref[k], dn,
                            preferred_element_type=jnp.float32)
            + lax.dot_general(h, whh_ref[k], dn,
                              preferred_element_type=jnp.float32)
            + bg[k]
            for k in range(4)
        ]
        i_g = _sigmoid(g[0])
        f_g = _sigmoid(g[1])
        g_g = jnp.tanh(g[2])
        o_g = _sigmoid(g[3])
        c = f_g * c + i_g * g_g
        h = o_g * jnp.tanh(c)
    out_ref[...] = (
        jnp.dot(h, wfc_ref[...], preferred_element_type=jnp.float32)
        + bfc_ref[...])


def _final(s2, y2, deg3, egof, b2r, wih4, whh4, bih4, bhh4, wfc, bfcr):
    return pl.pallas_call(
        _final_body,
        grid=(N // BN2,),
        in_specs=[
            pl.BlockSpec((T, BN2, H), lambda i: (0, i, 0)),
            pl.BlockSpec((T, BN2, H), lambda i: (0, i, 0)),
            pl.BlockSpec((T, BN2, 1), lambda i: (0, i, 0)),
            pl.BlockSpec((T, BN2, 1), lambda i: (0, i, 0)),
            pl.BlockSpec((1, H), lambda i: (0, 0)),
            pl.BlockSpec((4, H, H), lambda i: (0, 0, 0)),
            pl.BlockSpec((4, H, H), lambda i: (0, 0, 0)),
            pl.BlockSpec((4, H), lambda i: (0, 0)),
            pl.BlockSpec((4, H), lambda i: (0, 0)),
            pl.BlockSpec((H, D_OUT), lambda i: (0, 0)),
            pl.BlockSpec((1, D_OUT), lambda i: (0, 0)),
        ],
        out_specs=pl.BlockSpec((BN2, D_OUT), lambda i: (i, 0)),
        out_shape=jax.ShapeDtypeStruct((N, D_OUT), jnp.float32),
    )(s2, y2, deg3, egof, b2r, wih4, whh4, bih4, bhh4, wfc, bfcr)


def kernel(x, edge_indices, ego_mask, W1, b1, W2, b2, W_ih, W_hh, b_ih, b_hh,
           W_fc, b_fc):
    src_flat = edge_indices[:, 0, :].reshape(T * E)
    dst_flat = edge_indices[:, 1, :].reshape(T * E)

    deg = _make_deg_kernel()(dst_flat)
    deg3 = deg.reshape(T, NPAD)[:, :N].reshape(T, N, 1)

    y1 = _mm1(x, W1, deg3)
    s1 = _make_agg_kernel()(y1.reshape(T * N, H), src_flat, dst_flat)
    s1 = s1.reshape(T, NPAD, H)[:, :N]

    y2 = _mm2(s1, y1, deg3, b1.reshape(1, H), W2)
    s2 = _make_agg_kernel()(y2.reshape(T * N, H), src_flat, dst_flat)
    s2 = s2.reshape(T, NPAD, H)[:, :N]

    egof = jnp.transpose(ego_mask, (1, 0, 2)).reshape(T, N, 1).astype(
        jnp.float32)

    out = _final(s2, y2, deg3, egof, b2.reshape(1, H),
                 W_ih.reshape(4, H, H), W_hh.reshape(4, H, H),
                 b_ih.reshape(4, H), b_hh.reshape(4, H),
                 W_fc, b_fc.reshape(1, D_OUT))
    return out.reshape(B, 400, D_OUT)


# speedup vs baseline: 35.7928x; 3.0406x over previous
import functools

import jax
import jax.numpy as jnp
from jax import lax
from jax.experimental import pallas as pl
from jax.experimental.pallas import tpu as pltpu
from jax.experimental.pallas import tpu_sc as plsc

N = 10000
T = 20
E = 320000
D_IN = 128
H = 64
D_OUT = 128
B = 25

NC = 2
NS = 16
ROWS_PER_TILE = 640
NPAD = NS * ROWS_PER_TILE
EDGES_PER_TILE = E // NS
CHUNK = 80
NCHUNKS = EDGES_PER_TILE // CHUNK
T_PER_SC = T // NC

NG_D = 25
NG = 5


@functools.cache
def _make_deg_kernel():
    return pl.kernel(
        _deg_body,
        out_type=jax.ShapeDtypeStruct((T * NPAD,), jnp.float32),
        mesh=plsc.VectorSubcoreMesh(core_axis_name="c", subcore_axis_name="s"),
        compiler_params=pltpu.CompilerParams(use_tc_tiling_on_sc=False),
        scratch_types=[
            pltpu.VMEM((ROWS_PER_TILE,), jnp.float32),
            pltpu.VMEM((NCHUNKS, CHUNK), jnp.int32),
            pltpu.VMEM_SHARED((NPAD,), jnp.float32),
            pltpu.SemaphoreType.DMA,
            pltpu.SemaphoreType.DMA,
        ],
    )


def _deg_body(dst_hbm, deg_hbm, ones_v, idx_v, acc, lsem, ssem):
    c = lax.axis_index("c")
    s = lax.axis_index("s")

    def fill_ones(i, _):
        ones_v[pl.ds(i * 16, 16)] = jnp.full((16,), 1.0, jnp.float32)
        return 0

    lax.fori_loop(0, ROWS_PER_TILE // 16, fill_ones, 0)

    my_rows = pl.ds(s * ROWS_PER_TILE, ROWS_PER_TILE)
    for j in range(T_PER_SC):
        t = j * NC + c
        row0 = (t * E + s * EDGES_PER_TILE) // CHUNK
        cp = pltpu.async_copy(dst_hbm.at[pl.ds(row0, NCHUNKS)], idx_v, lsem)
        pltpu.sync_copy(ones_v, acc.at[my_rows])
        cp.wait()
        plsc.subcore_barrier()

        def group(g, _):
            cps = [
                pltpu.async_copy(ones_v.at[pl.ds(0, CHUNK)],
                                 acc.at[idx_v.at[g * NG_D + b]], ssem,
                                 add=True)
                for b in range(NG_D)
            ]
            for c2 in cps:
                c2.wait()
            return 0

        lax.fori_loop(0, NCHUNKS // NG_D, group, 0)
        plsc.subcore_barrier()
        pltpu.sync_copy(acc.at[my_rows],
                        deg_hbm.at[pl.ds(t * NPAD + s * ROWS_PER_TILE,
                                         ROWS_PER_TILE)])


@functools.cache
def _make_agg_kernel():
    return pl.kernel(
        _agg_body,
        out_type=jax.ShapeDtypeStruct((T * NPAD, H), jnp.float32),
        mesh=plsc.VectorSubcoreMesh(core_axis_name="c", subcore_axis_name="s"),
        compiler_params=pltpu.CompilerParams(use_tc_tiling_on_sc=False),
        scratch_types=[
            pltpu.VMEM((NCHUNKS, CHUNK), jnp.int32),
            pltpu.VMEM((NCHUNKS, CHUNK), jnp.int32),
            pltpu.VMEM((NG, CHUNK, H), jnp.float32),
            pltpu.VMEM((CHUNK, H), jnp.float32),
            pltpu.VMEM_SHARED((NPAD, H), jnp.float32),
            pltpu.SemaphoreType.DMA,
            pltpu.SemaphoreType.DMA,
            pltpu.SemaphoreType.DMA,
        ],
    )


def _agg_body(y_hbm, src_hbm, dst_hbm, s_hbm, idx_s, idx_d, rows_v, zeros_v,
              acc, lsem, gsem, ssem):
    c = lax.axis_index("c")
    s = lax.axis_index("s")

    def fill_zeros(i, _):
        zeros_v[i // 4, pl.ds((i % 4) * 16, 16)] = jnp.zeros((16,), jnp.float32)
        return 0

    lax.fori_loop(0, CHUNK * (H // 16), fill_zeros, 0)

    for j in range(T_PER_SC):
        t = j * NC + c
        row0 = (t * E + s * EDGES_PER_TILE) // CHUNK
        cp_s = pltpu.async_copy(src_hbm.at[pl.ds(row0, NCHUNKS)], idx_s, lsem)
        cp_d = pltpu.async_copy(dst_hbm.at[pl.ds(row0, NCHUNKS)], idx_d, lsem)
        for q in range(ROWS_PER_TILE // CHUNK):
            pltpu.sync_copy(
                zeros_v, acc.at[pl.ds(s * ROWS_PER_TILE + q * CHUNK, CHUNK)])
        cp_s.wait()
        cp_d.wait()
        plsc.subcore_barrier()

        y_t = y_hbm.at[pl.ds(t * N, N)]

        def group(g, _):
            cps = [
                pltpu.async_copy(y_t.at[idx_s.at[g * NG + b]], rows_v.at[b],
                                 gsem)
                for b in range(NG)
            ]
            for c2 in cps:
                c2.wait()
            cps = [
                pltpu.async_copy(rows_v.at[b], acc.at[idx_d.at[g * NG + b]],
                                 ssem, add=True)
                for b in range(NG)
            ]
            for c2 in cps:
                c2.wait()
            return 0

        lax.fori_loop(0, NCHUNKS // NG, group, 0)
        plsc.subcore_barrier()
        pltpu.sync_copy(
            acc.at[pl.ds(s * ROWS_PER_TILE, ROWS_PER_TILE)],
            s_hbm.at[pl.ds(t * NPAD + s * ROWS_PER_TILE, ROWS_PER_TILE)])


BN = 2000
NB = N // BN
BN2 = 400


def _mm1_body(x_ref, w_ref, deg_ref, y_ref):
    dinv = lax.rsqrt(deg_ref[0])
    y_ref[0] = (
        jnp.dot(x_ref[0], w_ref[...], preferred_element_type=jnp.float32)
        * dinv)


def _mm1(x, w1, deg3):
    return pl.pallas_call(
        _mm1_body,
        grid=(T, NB),
        in_specs=[
            pl.BlockSpec((1, BN, D_IN), lambda t, i: (t, i, 0)),
            pl.BlockSpec((D_IN, H), lambda t, i: (0, 0)),
            pl.BlockSpec((1, BN, 1), lambda t, i: (t, i, 0)),
        ],
        out_specs=pl.BlockSpec((1, BN, H), lambda t, i: (t, i, 0)),
        out_shape=jax.ShapeDtypeStruct((T, N, H), jnp.float32),
    )(x, w1, deg3)


def _mm2_body(s_ref, y_ref, deg_ref, b_ref, w_ref, out_ref):
    dinv = lax.rsqrt(deg_ref[0])
    z = jnp.maximum(dinv * (s_ref[0] + y_ref[0]) + b_ref[...], 0.0)
    out_ref[0] = (
        jnp.dot(z, w_ref[...], preferred_element_type=jnp.float32) * dinv)


def _mm2(s1, y1, deg3, b1r, w2):
    return pl.pallas_call(
        _mm2_body,
        grid=(T, NB),
        in_specs=[
            pl.BlockSpec((1, BN, H), lambda t, i: (t, i, 0)),
            pl.BlockSpec((1, BN, H), lambda t, i: (t, i, 0)),
            pl.BlockSpec((1, BN, 1), lambda t, i: (t, i, 0)),
            pl.BlockSpec((1, H), lambda t, i: (0, 0)),
            pl.BlockSpec((H, H), lambda t, i: (0, 0)),
        ],
        out_specs=pl.BlockSpec((1, BN, H), lambda t, i: (t, i, 0)),
        out_shape=jax.ShapeDtypeStruct((T, N, H), jnp.float32),
    )(s1, y1, deg3, b1r, w2)


def _sigmoid(x):
    return 1.0 / (1.0 + jnp.exp(-x))


def _final_body(s_ref, y_ref, deg_ref, m_ref, b2_ref, wih_ref, whh_ref,
                bih_ref, bhh_ref, wfc_ref, bfc_ref, out_ref):
    h = jnp.zeros((BN2, H), jnp.float32)
    c = jnp.zeros((BN2, H), jnp.float32)
    bg = bih_ref[...] + bhh_ref[...]
    dn = (((1,), (1,)), ((), ()))
    for t in range(T):
        dinv = lax.rsqrt(deg_ref[t])
        keep = 1.0 - m_ref[t]
        xo = (dinv * (s_ref[t] + y_ref[t]) + b2_ref[...]) * keep
        g = [
            lax.dot_general(xo, wih_ref[k], dn,
                            preferred_element_type=jnp.float32)
            + lax.dot_general(h, whh_ref[k], dn,
                              preferred_element_type=jnp.float32)
            + bg[k]
            for k in range(4)
        ]
        i_g = _sigmoid(g[0])
        f_g = _sigmoid(g[1])
        g_g = jnp.tanh(g[2])
        o_g = _sigmoid(g[3])
        c = f_g * c + i_g * g_g
        h = o_g * jnp.tanh(c)
    out_ref[...] = (
        jnp.dot(h, wfc_ref[...], preferred_element_type=jnp.float32)
        + bfc_ref[...])


def _final(s2, y2, deg3, egof, b2r, wih4, whh4, bih4, bhh4, wfc, bfcr):
    return pl.pallas_call(
        _final_body,
        grid=(N // BN2,),
        in_specs=[
            pl.BlockSpec((T, BN2, H), lambda i: (0, i, 0)),
            pl.BlockSpec((T, BN2, H), lambda i: (0, i, 0)),
            pl.BlockSpec((T, BN2, 1), lambda i: (0, i, 0)),
            pl.BlockSpec((T, BN2, 1), lambda i: (0, i, 0)),
            pl.BlockSpec((1, H), lambda i: (0, 0)),
            pl.BlockSpec((4, H, H), lambda i: (0, 0, 0)),
            pl.BlockSpec((4, H, H), lambda i: (0, 0, 0)),
            pl.BlockSpec((4, H), lambda i: (0, 0)),
            pl.BlockSpec((4, H), lambda i: (0, 0)),
            pl.BlockSpec((H, D_OUT), lambda i: (0, 0)),
            pl.BlockSpec((1, D_OUT), lambda i: (0, 0)),
        ],
        out_specs=pl.BlockSpec((BN2, D_OUT), lambda i: (i, 0)),
        out_shape=jax.ShapeDtypeStruct((N, D_OUT), jnp.float32),
    )(s2, y2, deg3, egof, b2r, wih4, whh4, bih4, bhh4, wfc, bfcr)


def kernel(x, edge_indices, ego_mask, W1, b1, W2, b2, W_ih, W_hh, b_ih, b_hh,
           W_fc, b_fc):
    src_flat = edge_indices[:, 0, :].reshape(T * E // CHUNK, CHUNK)
    dst_flat = edge_indices[:, 1, :].reshape(T * E // CHUNK, CHUNK)

    deg = _make_deg_kernel()(dst_flat)
    deg3 = deg.reshape(T, NPAD)[:, :N].reshape(T, N, 1)

    y1 = _mm1(x, W1, deg3)
    s1 = _make_agg_kernel()(y1.reshape(T * N, H), src_flat, dst_flat)
    s1 = s1.reshape(T, NPAD, H)[:, :N]

    y2 = _mm2(s1, y1, deg3, b1.reshape(1, H), W2)
    s2 = _make_agg_kernel()(y2.reshape(T * N, H), src_flat, dst_flat)
    s2 = s2.reshape(T, NPAD, H)[:, :N]

    egof = jnp.transpose(ego_mask, (1, 0, 2)).reshape(T, N, 1).astype(
        jnp.float32)

    out = _final(s2, y2, deg3, egof, b2.reshape(1, H),
                 W_ih.reshape(4, H, H), W_hh.reshape(4, H, H),
                 b_ih.reshape(4, H), b_hh.reshape(4, H),
                 W_fc, b_fc.reshape(1, D_OUT))
    return out.reshape(B, 400, D_OUT)


# speedup vs baseline: 36.5158x; 1.0202x over previous
import functools

import jax
import jax.numpy as jnp
from jax import lax
from jax.experimental import pallas as pl
from jax.experimental.pallas import tpu as pltpu
from jax.experimental.pallas import tpu_sc as plsc

N = 10000
T = 20
E = 320000
D_IN = 128
H = 64
D_OUT = 128
B = 25

NC = 2
NS = 16
ROWS_PER_TILE = 640
NPAD = NS * ROWS_PER_TILE
EDGES_PER_TILE = E // NS
CHUNK = 80
NCHUNKS = EDGES_PER_TILE // CHUNK
T_PER_SC = T // NC

NG_D = 25
NG = 5


@functools.cache
def _make_deg_kernel():
    return pl.kernel(
        _deg_body,
        out_type=jax.ShapeDtypeStruct((T * NPAD,), jnp.float32),
        mesh=plsc.VectorSubcoreMesh(core_axis_name="c", subcore_axis_name="s"),
        compiler_params=pltpu.CompilerParams(use_tc_tiling_on_sc=False),
        scratch_types=[
            pltpu.VMEM((ROWS_PER_TILE,), jnp.float32),
            pltpu.VMEM((NCHUNKS, CHUNK), jnp.int32),
            pltpu.VMEM_SHARED((NPAD,), jnp.float32),
            pltpu.SemaphoreType.DMA,
            pltpu.SemaphoreType.DMA,
        ],
    )


def _deg_body(dst_hbm, deg_hbm, ones_v, idx_v, acc, lsem, ssem):
    c = lax.axis_index("c")
    s = lax.axis_index("s")

    def fill_ones(i, _):
        ones_v[pl.ds(i * 16, 16)] = jnp.full((16,), 1.0, jnp.float32)
        return 0

    lax.fori_loop(0, ROWS_PER_TILE // 16, fill_ones, 0)

    my_rows = pl.ds(s * ROWS_PER_TILE, ROWS_PER_TILE)
    for j in range(T_PER_SC):
        t = j * NC + c
        row0 = (t * E + s * EDGES_PER_TILE) // CHUNK
        cp = pltpu.async_copy(dst_hbm.at[pl.ds(row0, NCHUNKS)], idx_v, lsem)
        pltpu.sync_copy(ones_v, acc.at[my_rows])
        cp.wait()
        plsc.subcore_barrier()

        def group(g, _):
            cps = [
                pltpu.async_copy(ones_v.at[pl.ds(0, CHUNK)],
                                 acc.at[idx_v.at[g * NG_D + b]], ssem,
                                 add=True)
                for b in range(NG_D)
            ]
            for c2 in cps:
                c2.wait()
            return 0

        lax.fori_loop(0, NCHUNKS // NG_D, group, 0)
        plsc.subcore_barrier()
        pltpu.sync_copy(acc.at[my_rows],
                        deg_hbm.at[pl.ds(t * NPAD + s * ROWS_PER_TILE,
                                         ROWS_PER_TILE)])


@functools.cache
def _make_agg_kernel():
    return pl.kernel(
        _agg_body,
        out_type=jax.ShapeDtypeStruct((T * NPAD, H), jnp.float32),
        mesh=plsc.VectorSubcoreMesh(core_axis_name="c", subcore_axis_name="s"),
        compiler_params=pltpu.CompilerParams(use_tc_tiling_on_sc=False),
        scratch_types=[
            pltpu.VMEM((NCHUNKS, CHUNK), jnp.int32),
            pltpu.VMEM((NCHUNKS, CHUNK), jnp.int32),
            pltpu.VMEM((NG, CHUNK, H), jnp.float32),
            pltpu.VMEM((CHUNK, H), jnp.float32),
            pltpu.VMEM_SHARED((NPAD, H), jnp.float32),
            pltpu.SemaphoreType.DMA,
            pltpu.SemaphoreType.DMA,
            pltpu.SemaphoreType.DMA,
        ],
    )


def _agg_body(y_hbm, src_hbm, dst_hbm, s_hbm, idx_s, idx_d, rows_v, zeros_v,
              acc, lsem, gsem, ssem):
    c = lax.axis_index("c")
    s = lax.axis_index("s")

    def fill_zeros(i, _):
        zeros_v[i // 4, pl.ds((i % 4) * 16, 16)] = jnp.zeros((16,), jnp.float32)
        return 0

    lax.fori_loop(0, CHUNK * (H // 16), fill_zeros, 0)

    for j in range(T_PER_SC):
        t = j * NC + c
        row0 = (t * E + s * EDGES_PER_TILE) // CHUNK
        cp_s = pltpu.async_copy(src_hbm.at[pl.ds(row0, NCHUNKS)], idx_s, lsem)
        cp_d = pltpu.async_copy(dst_hbm.at[pl.ds(row0, NCHUNKS)], idx_d, lsem)
        for q in range(ROWS_PER_TILE // CHUNK):
            pltpu.sync_copy(
                zeros_v, acc.at[pl.ds(s * ROWS_PER_TILE + q * CHUNK, CHUNK)])
        cp_s.wait()
        cp_d.wait()
        plsc.subcore_barrier()

        y_t = y_hbm.at[pl.ds(t * N, N)]

        def group(g, _):
            cps = [
                pltpu.async_copy(y_t.at[idx_s.at[g * NG + b]], rows_v.at[b],
                                 gsem)
                for b in range(NG)
            ]
            for c2 in cps:
                c2.wait()
            cps = [
                pltpu.async_copy(rows_v.at[b], acc.at[idx_d.at[g * NG + b]],
                                 ssem, add=True)
                for b in range(NG)
            ]
            for c2 in cps:
                c2.wait()
            return 0

        lax.fori_loop(0, NCHUNKS // NG, group, 0)
        plsc.subcore_barrier()
        pltpu.sync_copy(
            acc.at[pl.ds(s * ROWS_PER_TILE, ROWS_PER_TILE)],
            s_hbm.at[pl.ds(t * NPAD + s * ROWS_PER_TILE, ROWS_PER_TILE)])


BN = 2000
NB = N // BN
BN2 = 400


def _mm1_body(x_ref, w_ref, deg_ref, y_ref):
    dinv = lax.rsqrt(deg_ref[0])
    y_ref[0] = (
        jnp.dot(x_ref[0], w_ref[...], preferred_element_type=jnp.float32)
        * dinv)


def _mm1(x, w1, deg3):
    return pl.pallas_call(
        _mm1_body,
        grid=(T, NB),
        in_specs=[
            pl.BlockSpec((1, BN, D_IN), lambda t, i: (t, i, 0)),
            pl.BlockSpec((D_IN, H), lambda t, i: (0, 0)),
            pl.BlockSpec((1, BN, 1), lambda t, i: (t, i, 0)),
        ],
        out_specs=pl.BlockSpec((1, BN, H), lambda t, i: (t, i, 0)),
        out_shape=jax.ShapeDtypeStruct((T, N, H), jnp.float32),
    )(x, w1, deg3)


def _mm2_body(s_ref, y_ref, deg_ref, b_ref, w_ref, out_ref):
    dinv = lax.rsqrt(deg_ref[0])
    z = jnp.maximum(dinv * (s_ref[0] + y_ref[0]) + b_ref[...], 0.0)
    out_ref[0] = (
        jnp.dot(z, w_ref[...], preferred_element_type=jnp.float32) * dinv)


def _mm2(s1, y1, deg3, b1r, w2):
    return pl.pallas_call(
        _mm2_body,
        grid=(T, NB),
        in_specs=[
            pl.BlockSpec((1, BN, H), lambda t, i: (t, i, 0)),
            pl.BlockSpec((1, BN, H), lambda t, i: (t, i, 0)),
            pl.BlockSpec((1, BN, 1), lambda t, i: (t, i, 0)),
            pl.BlockSpec((1, H), lambda t, i: (0, 0)),
            pl.BlockSpec((H, H), lambda t, i: (0, 0)),
        ],
        out_specs=pl.BlockSpec((1, BN, H), lambda t, i: (t, i, 0)),
        out_shape=jax.ShapeDtypeStruct((T, N, H), jnp.float32),
    )(s1, y1, deg3, b1r, w2)


def _sigmoid(x):
    return 1.0 / (1.0 + jnp.exp(-x))


def _final_body(s_ref, y_ref, deg_ref, m_ref, b2_ref, wih_ref, whh_ref,
                bih_ref, bhh_ref, wfc_ref, bfc_ref, out_ref):
    h = jnp.zeros((BN2, H), jnp.float32)
    c = jnp.zeros((BN2, H), jnp.float32)
    bg = bih_ref[...] + bhh_ref[...]
    dn = (((1,), (1,)), ((), ()))
    for t in range(T):
        dinv = lax.rsqrt(deg_ref[t])
        keep = 1.0 - m_ref[t]
        xo = (dinv * (s_ref[t] + y_ref[t]) + b2_ref[...]) * keep
        g = [
            lax.dot_general(xo, wih_ref[k], dn,
                            preferred_element_type=jnp.float32)
            + lax.dot_general(h, whh_ref[k], dn,
                              preferred_element_type=jnp.float32)
            + bg[k]
            for k in range(4)
        ]
        i_g = _sigmoid(g[0])
        f_g = _sigmoid(g[1])
        g_g = jnp.tanh(g[2])
        o_g = _sigmoid(g[3])
        c = f_g * c + i_g * g_g
        h = o_g * jnp.tanh(c)
    out_ref[...] = (
        jnp.dot(h, wfc_ref[...], preferred_element_type=jnp.float32)
        + bfc_ref[...])


def _final(s2, y2, deg3, egof, b2r, wih4, whh4, bih4, bhh4, wfc, bfcr):
    return pl.pallas_call(
        _final_body,
        grid=(N // BN2,),
        in_specs=[
            pl.BlockSpec((T, BN2, H), lambda i: (0, i, 0)),
            pl.BlockSpec((T, BN2, H), lambda i: (0, i, 0)),
            pl.BlockSpec((T, BN2, 1), lambda i: (0, i, 0)),
            pl.BlockSpec((T, BN2, 1), lambda i: (0, i, 0)),
            pl.BlockSpec((1, H), lambda i: (0, 0)),
            pl.BlockSpec((4, H, H), lambda i: (0, 0, 0)),
            pl.BlockSpec((4, H, H), lambda i: (0, 0, 0)),
            pl.BlockSpec((4, H), lambda i: (0, 0)),
            pl.BlockSpec((4, H), lambda i: (0, 0)),
            pl.BlockSpec((H, D_OUT), lambda i: (0, 0)),
            pl.BlockSpec((1, D_OUT), lambda i: (0, 0)),
        ],
        out_specs=pl.BlockSpec((BN2, D_OUT), lambda i: (i, 0)),
        out_shape=jax.ShapeDtypeStruct((N, D_OUT), jnp.float32),
    )(s2, y2, deg3, egof, b2r, wih4, whh4, bih4, bhh4, wfc, bfcr)


def kernel(x, edge_indices, ego_mask, W1, b1, W2, b2, W_ih, W_hh, b_ih, b_hh,
           W_fc, b_fc):
    src_flat = edge_indices[:, 0, :].reshape(T * E // CHUNK, CHUNK)
    dst_flat = edge_indices[:, 1, :].reshape(T * E // CHUNK, CHUNK)

    deg = _make_deg_kernel()(dst_flat)
    deg3 = deg.reshape(T, NPAD, 1)

    y1 = _mm1(x, W1, deg3)
    s1 = _make_agg_kernel()(y1.reshape(T * N, H), src_flat, dst_flat)
    s1 = s1.reshape(T, NPAD, H)

    y2 = _mm2(s1, y1, deg3, b1.reshape(1, H), W2)
    s2 = _make_agg_kernel()(y2.reshape(T * N, H), src_flat, dst_flat)
    s2 = s2.reshape(T, NPAD, H)

    egof = jnp.transpose(ego_mask, (1, 0, 2)).reshape(T, N, 1).astype(
        jnp.float32)

    out = _final(s2, y2, deg3, egof, b2.reshape(1, H),
                 W_ih.reshape(4, H, H), W_hh.reshape(4, H, H),
                 b_ih.reshape(4, H), b_hh.reshape(4, H),
                 W_fc, b_fc.reshape(1, D_OUT))
    return out.reshape(B, 400, D_OUT)


# speedup vs baseline: 38.6226x; 1.0577x over previous
import functools

import jax
import jax.numpy as jnp
from jax import lax
from jax.experimental import pallas as pl
from jax.experimental.pallas import tpu as pltpu
from jax.experimental.pallas import tpu_sc as plsc

N = 10000
T = 20
E = 320000
D_IN = 128
H = 64
D_OUT = 128
B = 25

NC = 2
NS = 16
ROWS_PER_TILE = 640
NPAD = NS * ROWS_PER_TILE
EDGES_PER_TILE = E // NS
CHUNK = 80
NCHUNKS = EDGES_PER_TILE // CHUNK
T_PER_SC = T // NC

NG_D = 25
NG = 10
PHASES = ((0, 12), (120, 13))
IDX_ROWS = 130


@functools.cache
def _make_deg_kernel():
    return pl.kernel(
        _deg_body,
        out_type=jax.ShapeDtypeStruct((T * NPAD,), jnp.float32),
        mesh=plsc.VectorSubcoreMesh(core_axis_name="c", subcore_axis_name="s"),
        compiler_params=pltpu.CompilerParams(use_tc_tiling_on_sc=False),
        scratch_types=[
            pltpu.VMEM((ROWS_PER_TILE,), jnp.float32),
            pltpu.VMEM((NCHUNKS, CHUNK), jnp.int32),
            pltpu.VMEM_SHARED((NPAD,), jnp.float32),
            pltpu.SemaphoreType.DMA,
            pltpu.SemaphoreType.DMA,
        ],
    )


def _deg_body(dst_hbm, deg_hbm, ones_v, idx_v, acc, lsem, ssem):
    c = lax.axis_index("c")
    s = lax.axis_index("s")

    def fill_ones(i, _):
        ones_v[pl.ds(i * 16, 16)] = jnp.full((16,), 1.0, jnp.float32)
        return 0

    lax.fori_loop(0, ROWS_PER_TILE // 16, fill_ones, 0)

    my_rows = pl.ds(s * ROWS_PER_TILE, ROWS_PER_TILE)
    for j in range(T_PER_SC):
        t = j * NC + c
        row0 = (t * E + s * EDGES_PER_TILE) // CHUNK
        cp = pltpu.async_copy(dst_hbm.at[pl.ds(row0, NCHUNKS)], idx_v, lsem)
        pltpu.sync_copy(ones_v, acc.at[my_rows])
        cp.wait()
        plsc.subcore_barrier()

        def group(g, _):
            cps = [
                pltpu.async_copy(ones_v.at[pl.ds(0, CHUNK)],
                                 acc.at[idx_v.at[g * NG_D + b]], ssem,
                                 add=True)
                for b in range(NG_D)
            ]
            for c2 in cps:
                c2.wait()
            return 0

        lax.fori_loop(0, NCHUNKS // NG_D, group, 0)
        plsc.subcore_barrier()
        pltpu.sync_copy(acc.at[my_rows],
                        deg_hbm.at[pl.ds(t * NPAD + s * ROWS_PER_TILE,
                                         ROWS_PER_TILE)])


@functools.cache
def _make_agg_kernel():
    return pl.kernel(
        _agg_body,
        out_type=jax.ShapeDtypeStruct((T * NPAD, H), jnp.float32),
        mesh=plsc.VectorSubcoreMesh(core_axis_name="c", subcore_axis_name="s"),
        compiler_params=pltpu.CompilerParams(use_tc_tiling_on_sc=False),
        scratch_types=[
            pltpu.VMEM((IDX_ROWS, CHUNK), jnp.int32),
            pltpu.VMEM((IDX_ROWS, CHUNK), jnp.int32),
            pltpu.VMEM((NG, CHUNK, H), jnp.float32),
            pltpu.VMEM((CHUNK, H), jnp.float32),
            pltpu.VMEM_SHARED((NPAD, H), jnp.float32),
            pltpu.SemaphoreType.DMA,
            pltpu.SemaphoreType.DMA,
            pltpu.SemaphoreType.DMA,
        ],
    )


def _agg_body(y_hbm, src_hbm, dst_hbm, s_hbm, idx_s, idx_d, rows_v, zeros_v,
              acc, lsem, gsem, ssem):
    c = lax.axis_index("c")
    s = lax.axis_index("s")

    def fill_zeros(i, _):
        zeros_v[i // 4, pl.ds((i % 4) * 16, 16)] = jnp.zeros((16,), jnp.float32)
        return 0

    lax.fori_loop(0, CHUNK * (H // 16), fill_zeros, 0)

    for j in range(T_PER_SC):
        t = j * NC + c
        row0 = (t * E + s * EDGES_PER_TILE) // CHUNK
        first = True
        y_t = y_hbm.at[pl.ds(t * N, N)]
        for c0, ngroups in PHASES:
            nrows = ngroups * NG
            cp_s = pltpu.async_copy(
                src_hbm.at[pl.ds(row0 + c0, nrows)],
                idx_s.at[pl.ds(0, nrows)], lsem)
            cp_d = pltpu.async_copy(
                dst_hbm.at[pl.ds(row0 + c0, nrows)],
                idx_d.at[pl.ds(0, nrows)], lsem)
            if first:
                for q in range(ROWS_PER_TILE // CHUNK):
                    pltpu.sync_copy(
                        zeros_v,
                        acc.at[pl.ds(s * ROWS_PER_TILE + q * CHUNK, CHUNK)])
            cp_s.wait()
            cp_d.wait()
            if first:
                plsc.subcore_barrier()
                first = False

            def group(g, _):
                cps = [
                    pltpu.async_copy(y_t.at[idx_s.at[g * NG + b]],
                                     rows_v.at[b], gsem)
                    for b in range(NG)
                ]
                for c2 in cps:
                    c2.wait()
                cps = [
                    pltpu.async_copy(rows_v.at[b],
                                     acc.at[idx_d.at[g * NG + b]],
                                     ssem, add=True)
                    for b in range(NG)
                ]
                for c2 in cps:
                    c2.wait()
                return 0

            lax.fori_loop(0, ngroups, group, 0)
        plsc.subcore_barrier()
        pltpu.sync_copy(
            acc.at[pl.ds(s * ROWS_PER_TILE, ROWS_PER_TILE)],
            s_hbm.at[pl.ds(t * NPAD + s * ROWS_PER_TILE, ROWS_PER_TILE)])


BN = 2000
NB = N // BN
BN2 = 400


def _mm1_body(x_ref, w_ref, deg_ref, y_ref):
    dinv = lax.rsqrt(deg_ref[0])
    y_ref[0] = (
        jnp.dot(x_ref[0], w_ref[...], preferred_element_type=jnp.float32)
        * dinv)


def _mm1(x, w1, deg3):
    return pl.pallas_call(
        _mm1_body,
        grid=(T, NB),
        in_specs=[
            pl.BlockSpec((1, BN, D_IN), lambda t, i: (t, i, 0)),
            pl.BlockSpec((D_IN, H), lambda t, i: (0, 0)),
            pl.BlockSpec((1, BN, 1), lambda t, i: (t, i, 0)),
        ],
        out_specs=pl.BlockSpec((1, BN, H), lambda t, i: (t, i, 0)),
        out_shape=jax.ShapeDtypeStruct((T, N, H), jnp.float32),
    )(x, w1, deg3)


def _mm2_body(s_ref, y_ref, deg_ref, b_ref, w_ref, out_ref):
    dinv = lax.rsqrt(deg_ref[0])
    z = jnp.maximum(dinv * (s_ref[0] + y_ref[0]) + b_ref[...], 0.0)
    out_ref[0] = (
        jnp.dot(z, w_ref[...], preferred_element_type=jnp.float32) * dinv)


def _mm2(s1, y1, deg3, b1r, w2):
    return pl.pallas_call(
        _mm2_body,
        grid=(T, NB),
        in_specs=[
            pl.BlockSpec((1, BN, H), lambda t, i: (t, i, 0)),
            pl.BlockSpec((1, BN, H), lambda t, i: (t, i, 0)),
            pl.BlockSpec((1, BN, 1), lambda t, i: (t, i, 0)),
            pl.BlockSpec((1, H), lambda t, i: (0, 0)),
            pl.BlockSpec((H, H), lambda t, i: (0, 0)),
        ],
        out_specs=pl.BlockSpec((1, BN, H), lambda t, i: (t, i, 0)),
        out_shape=jax.ShapeDtypeStruct((T, N, H), jnp.float32),
    )(s1, y1, deg3, b1r, w2)


def _sigmoid(x):
    return 1.0 / (1.0 + jnp.exp(-x))


def _final_body(s_ref, y_ref, deg_ref, m_ref, b2_ref, wih_ref, whh_ref,
                bih_ref, bhh_ref, wfc_ref, bfc_ref, out_ref):
    h = jnp.zeros((BN2, H), jnp.float32)
    c = jnp.zeros((BN2, H), jnp.float32)
    bg = bih_ref[...] + bhh_ref[...]
    dn = (((1,), (1,)), ((), ()))
    for t in range(T):
        dinv = lax.rsqrt(deg_ref[t])
        keep = 1.0 - m_ref[t]
        xo = (dinv * (s_ref[t] + y_ref[t]) + b2_ref[...]) * keep
        g = [
            lax.dot_general(xo, wih_ref[k], dn,
                            preferred_element_type=jnp.float32)
            + lax.dot_general(h, whh_ref[k], dn,
                              preferred_element_type=jnp.float32)
            + bg[k]
            for k in range(4)
        ]
        i_g = _sigmoid(g[0])
        f_g = _sigmoid(g[1])
        g_g = jnp.tanh(g[2])
        o_g = _sigmoid(g[3])
        c = f_g * c + i_g * g_g
        h = o_g * jnp.tanh(c)
    out_ref[...] = (
        jnp.dot(h, wfc_ref[...], preferred_element_type=jnp.float32)
        + bfc_ref[...])


def _final(s2, y2, deg3, egof, b2r, wih4, whh4, bih4, bhh4, wfc, bfcr):
    return pl.pallas_call(
        _final_body,
        grid=(N // BN2,),
        in_specs=[
            pl.BlockSpec((T, BN2, H), lambda i: (0, i, 0)),
            pl.BlockSpec((T, BN2, H), lambda i: (0, i, 0)),
            pl.BlockSpec((T, BN2, 1), lambda i: (0, i, 0)),
            pl.BlockSpec((T, BN2, 1), lambda i: (0, i, 0)),
            pl.BlockSpec((1, H), lambda i: (0, 0)),
            pl.BlockSpec((4, H, H), lambda i: (0, 0, 0)),
            pl.BlockSpec((4, H, H), lambda i: (0, 0, 0)),
            pl.BlockSpec((4, H), lambda i: (0, 0)),
            pl.BlockSpec((4, H), lambda i: (0, 0)),
            pl.BlockSpec((H, D_OUT), lambda i: (0, 0)),
            pl.BlockSpec((1, D_OUT), lambda i: (0, 0)),
        ],
        out_specs=pl.BlockSpec((BN2, D_OUT), lambda i: (i, 0)),
        out_shape=jax.ShapeDtypeStruct((N, D_OUT), jnp.float32),
    )(s2, y2, deg3, egof, b2r, wih4, whh4, bih4, bhh4, wfc, bfcr)


def kernel(x, edge_indices, ego_mask, W1, b1, W2, b2, W_ih, W_hh, b_ih, b_hh,
           W_fc, b_fc):
    src_flat = edge_indices[:, 0, :].reshape(T * E // CHUNK, CHUNK)
    dst_flat = edge_indices[:, 1, :].reshape(T * E // CHUNK, CHUNK)

    deg = _make_deg_kernel()(dst_flat)
    deg3 = deg.reshape(T, NPAD, 1)

    y1 = _mm1(x, W1, deg3)
    s1 = _make_agg_kernel()(y1.reshape(T * N, H), src_flat, dst_flat)
    s1 = s1.reshape(T, NPAD, H)

    y2 = _mm2(s1, y1, deg3, b1.reshape(1, H), W2)
    s2 = _make_agg_kernel()(y2.reshape(T * N, H), src_flat, dst_flat)
    s2 = s2.reshape(T, NPAD, H)

    egof = jnp.transpose(ego_mask, (1, 0, 2)).reshape(T, N, 1).astype(
        jnp.float32)

    out = _final(s2, y2, deg3, egof, b2.reshape(1, H),
                 W_ih.reshape(4, H, H), W_hh.reshape(4, H, H),
                 b_ih.reshape(4, H), b_hh.reshape(4, H),
                 W_fc, b_fc.reshape(1, D_OUT))
    return out.reshape(B, 400, D_OUT)


# speedup vs baseline: 44.1799x; 1.1439x over previous
import functools

import jax
import jax.numpy as jnp
from jax import lax
from jax.experimental import pallas as pl
from jax.experimental.pallas import tpu as pltpu
from jax.experimental.pallas import tpu_sc as plsc

N = 10000
T = 20
E = 320000
D_IN = 128
H = 64
D_OUT = 128
B = 25

NC = 2
NS = 16
ROWS_PER_TILE = 640
NPAD = NS * ROWS_PER_TILE
EDGES_PER_TILE = E // NS
CHUNK = 80
NCHUNKS = EDGES_PER_TILE // CHUNK
T_PER_SC = T // NC

NG_D = 25
NG = 5
PHASES = ((0, 12), (120, 13))
IDX_ROWS = 130


@functools.cache
def _make_deg_kernel():
    return pl.kernel(
        _deg_body,
        out_type=jax.ShapeDtypeStruct((T * NPAD,), jnp.float32),
        mesh=plsc.VectorSubcoreMesh(core_axis_name="c", subcore_axis_name="s"),
        compiler_params=pltpu.CompilerParams(use_tc_tiling_on_sc=False),
        scratch_types=[
            pltpu.VMEM((ROWS_PER_TILE,), jnp.float32),
            pltpu.VMEM((NCHUNKS, CHUNK), jnp.int32),
            pltpu.VMEM_SHARED((NPAD,), jnp.float32),
            pltpu.SemaphoreType.DMA,
            pltpu.SemaphoreType.DMA,
        ],
    )


def _deg_body(dst_hbm, deg_hbm, ones_v, idx_v, acc, lsem, ssem):
    c = lax.axis_index("c")
    s = lax.axis_index("s")

    def fill_ones(i, _):
        ones_v[pl.ds(i * 16, 16)] = jnp.full((16,), 1.0, jnp.float32)
        return 0

    lax.fori_loop(0, ROWS_PER_TILE // 16, fill_ones, 0)

    my_rows = pl.ds(s * ROWS_PER_TILE, ROWS_PER_TILE)
    for j in range(T_PER_SC):
        t = j * NC + c
        row0 = (t * E + s * EDGES_PER_TILE) // CHUNK
        cp = pltpu.async_copy(dst_hbm.at[pl.ds(row0, NCHUNKS)], idx_v, lsem)
        pltpu.sync_copy(ones_v, acc.at[my_rows])
        cp.wait()
        plsc.subcore_barrier()

        def group(g, _):
            cps = [
                pltpu.async_copy(ones_v.at[pl.ds(0, CHUNK)],
                                 acc.at[idx_v.at[g * NG_D + b]], ssem,
                                 add=True)
                for b in range(NG_D)
            ]
            for c2 in cps:
                c2.wait()
            return 0

        lax.fori_loop(0, NCHUNKS // NG_D, group, 0)
        plsc.subcore_barrier()
        pltpu.sync_copy(acc.at[my_rows],
                        deg_hbm.at[pl.ds(t * NPAD + s * ROWS_PER_TILE,
                                         ROWS_PER_TILE)])


@functools.cache
def _make_agg_kernel():
    return pl.kernel(
        _agg_body,
        out_type=jax.ShapeDtypeStruct((T * NPAD, H), jnp.float32),
        mesh=plsc.VectorSubcoreMesh(core_axis_name="c", subcore_axis_name="s"),
        compiler_params=pltpu.CompilerParams(use_tc_tiling_on_sc=False),
        scratch_types=[
            pltpu.VMEM((IDX_ROWS, CHUNK), jnp.int32),
            pltpu.VMEM((IDX_ROWS, CHUNK), jnp.int32),
            pltpu.VMEM((NG, CHUNK, H), jnp.float32),
            pltpu.VMEM((NG, CHUNK, H), jnp.float32),
            pltpu.VMEM((CHUNK, H), jnp.float32),
            pltpu.VMEM_SHARED((NPAD, H), jnp.float32),
            pltpu.SemaphoreType.DMA,
            pltpu.SemaphoreType.DMA,
            pltpu.SemaphoreType.DMA,
            pltpu.SemaphoreType.DMA,
            pltpu.SemaphoreType.DMA,
        ],
    )


def _agg_body(y_hbm, src_hbm, dst_hbm, s_hbm, idx_s, idx_d, rows_a, rows_b,
              zeros_v, acc, lsem, gsa, gsb, ssa, ssb):
    c = lax.axis_index("c")
    s = lax.axis_index("s")

    def fill_zeros(i, _):
        zeros_v[i // 4, pl.ds((i % 4) * 16, 16)] = jnp.zeros((16,), jnp.float32)
        return 0

    lax.fori_loop(0, CHUNK * (H // 16), fill_zeros, 0)

    for j in range(T_PER_SC):
        t = j * NC + c
        row0 = (t * E + s * EDGES_PER_TILE) // CHUNK
        first = True
        y_t = y_hbm.at[pl.ds(t * N, N)]

        def fire_gathers(buf, grp, sem, nrows):
            return [
                pltpu.async_copy(
                    y_t.at[idx_s.at[jnp.minimum(grp * NG + i, nrows - 1)]],
                    buf.at[i], sem)
                for i in range(NG)
            ]

        def fire_scatters(buf, grp, sem):
            return [
                pltpu.async_copy(buf.at[i], acc.at[idx_d.at[grp * NG + i]],
                                 sem, add=True)
                for i in range(NG)
            ]

        def drain_gathers(buf, sem):
            for i in range(NG):
                pltpu.make_async_copy(y_t.at[idx_s.at[0]], buf.at[i],
                                      sem).wait()

        def drain_scatters(buf, sem):
            for i in range(NG):
                pltpu.make_async_copy(buf.at[i], acc.at[idx_d.at[0]],
                                      sem).wait()

        for c0, npairs in PHASES:
            nrows = 2 * npairs * NG
            cp_s = pltpu.async_copy(
                src_hbm.at[pl.ds(row0 + c0, nrows)],
                idx_s.at[pl.ds(0, nrows)], lsem)
            cp_d = pltpu.async_copy(
                dst_hbm.at[pl.ds(row0 + c0, nrows)],
                idx_d.at[pl.ds(0, nrows)], lsem)
            if first:
                for q in range(ROWS_PER_TILE // CHUNK):
                    pltpu.sync_copy(
                        zeros_v,
                        acc.at[pl.ds(s * ROWS_PER_TILE + q * CHUNK, CHUNK)])
            cp_s.wait()
            cp_d.wait()
            if first:
                plsc.subcore_barrier()
                first = False

            fire_gathers(rows_a, 0, gsa, nrows)

            def pair(p, _):
                ga = 2 * p
                gb = 2 * p + 1
                drain_gathers(rows_a, gsa)
                fire_gathers(rows_b, gb, gsb, nrows)
                fire_scatters(rows_a, ga, ssa)
                drain_gathers(rows_b, gsb)
                fire_scatters(rows_b, gb, ssb)
                drain_scatters(rows_a, ssa)
                fire_gathers(rows_a, ga + 2, gsa, nrows)
                drain_scatters(rows_b, ssb)
                return 0

            lax.fori_loop(0, npairs, pair, 0)
            drain_gathers(rows_a, gsa)
        plsc.subcore_barrier()
        pltpu.sync_copy(
            acc.at[pl.ds(s * ROWS_PER_TILE, ROWS_PER_TILE)],
            s_hbm.at[pl.ds(t * NPAD + s * ROWS_PER_TILE, ROWS_PER_TILE)])


BN = 2000
NB = N // BN
BN2 = 400


def _mm1_body(x_ref, w_ref, deg_ref, y_ref):
    dinv = lax.rsqrt(deg_ref[0])
    y_ref[0] = (
        jnp.dot(x_ref[0], w_ref[...], preferred_element_type=jnp.float32)
        * dinv)


def _mm1(x, w1, deg3):
    return pl.pallas_call(
        _mm1_body,
        grid=(T, NB),
        in_specs=[
            pl.BlockSpec((1, BN, D_IN), lambda t, i: (t, i, 0)),
            pl.BlockSpec((D_IN, H), lambda t, i: (0, 0)),
            pl.BlockSpec((1, BN, 1), lambda t, i: (t, i, 0)),
        ],
        out_specs=pl.BlockSpec((1, BN, H), lambda t, i: (t, i, 0)),
        out_shape=jax.ShapeDtypeStruct((T, N, H), jnp.float32),
    )(x, w1, deg3)


def _mm2_body(s_ref, y_ref, deg_ref, b_ref, w_ref, out_ref):
    dinv = lax.rsqrt(deg_ref[0])
    z = jnp.maximum(dinv * (s_ref[0] + y_ref[0]) + b_ref[...], 0.0)
    out_ref[0] = (
        jnp.dot(z, w_ref[...], preferred_element_type=jnp.float32) * dinv)


def _mm2(s1, y1, deg3, b1r, w2):
    return pl.pallas_call(
        _mm2_body,
        grid=(T, NB),
        in_specs=[
            pl.BlockSpec((1, BN, H), lambda t, i: (t, i, 0)),
            pl.BlockSpec((1, BN, H), lambda t, i: (t, i, 0)),
            pl.BlockSpec((1, BN, 1), lambda t, i: (t, i, 0)),
            pl.BlockSpec((1, H), lambda t, i: (0, 0)),
            pl.BlockSpec((H, H), lambda t, i: (0, 0)),
        ],
        out_specs=pl.BlockSpec((1, BN, H), lambda t, i: (t, i, 0)),
        out_shape=jax.ShapeDtypeStruct((T, N, H), jnp.float32),
    )(s1, y1, deg3, b1r, w2)


def _sigmoid(x):
    return 0.5 * (1.0 + jnp.tanh(0.5 * x))


def _final_body(s_ref, y_ref, deg_ref, m_ref, b2_ref, wih_ref, whh_ref,
                bih_ref, bhh_ref, wfc_ref, bfc_ref, out_ref):
    h = jnp.zeros((BN2, H), jnp.float32)
    c = jnp.zeros((BN2, H), jnp.float32)
    bg = bih_ref[...] + bhh_ref[...]
    dn = (((1,), (1,)), ((), ()))
    for t in range(T):
        dinv = lax.rsqrt(deg_ref[t])
        keep = 1.0 - m_ref[t]
        xo = (dinv * (s_ref[t] + y_ref[t]) + b2_ref[...]) * keep
        g = [
            lax.dot_general(xo, wih_ref[k], dn,
                            preferred_element_type=jnp.float32)
            + lax.dot_general(h, whh_ref[k], dn,
                              preferred_element_type=jnp.float32)
            + bg[k]
            for k in range(4)
        ]
        i_g = _sigmoid(g[0])
        f_g = _sigmoid(g[1])
        g_g = jnp.tanh(g[2])
        o_g = _sigmoid(g[3])
        c = f_g * c + i_g * g_g
        h = o_g * jnp.tanh(c)
    out_ref[...] = (
        jnp.dot(h, wfc_ref[...], preferred_element_type=jnp.float32)
        + bfc_ref[...])


def _final(s2, y2, deg3, egof, b2r, wih4, whh4, bih4, bhh4, wfc, bfcr):
    return pl.pallas_call(
        _final_body,
        grid=(N // BN2,),
        in_specs=[
            pl.BlockSpec((T, BN2, H), lambda i: (0, i, 0)),
            pl.BlockSpec((T, BN2, H), lambda i: (0, i, 0)),
            pl.BlockSpec((T, BN2, 1), lambda i: (0, i, 0)),
            pl.BlockSpec((T, BN2, 1), lambda i: (0, i, 0)),
            pl.BlockSpec((1, H), lambda i: (0, 0)),
            pl.BlockSpec((4, H, H), lambda i: (0, 0, 0)),
            pl.BlockSpec((4, H, H), lambda i: (0, 0, 0)),
            pl.BlockSpec((4, H), lambda i: (0, 0)),
            pl.BlockSpec((4, H), lambda i: (0, 0)),
            pl.BlockSpec((H, D_OUT), lambda i: (0, 0)),
            pl.BlockSpec((1, D_OUT), lambda i: (0, 0)),
        ],
        out_specs=pl.BlockSpec((BN2, D_OUT), lambda i: (i, 0)),
        out_shape=jax.ShapeDtypeStruct((N, D_OUT), jnp.float32),
    )(s2, y2, deg3, egof, b2r, wih4, whh4, bih4, bhh4, wfc, bfcr)


def kernel(x, edge_indices, ego_mask, W1, b1, W2, b2, W_ih, W_hh, b_ih, b_hh,
           W_fc, b_fc):
    src_flat = edge_indices[:, 0, :].reshape(T * E // CHUNK, CHUNK)
    dst_flat = edge_indices[:, 1, :].reshape(T * E // CHUNK, CHUNK)

    deg = _make_deg_kernel()(dst_flat)
    deg3 = deg.reshape(T, NPAD, 1)

    y1 = _mm1(x, W1, deg3)
    s1 = _make_agg_kernel()(y1.reshape(T * N, H), src_flat, dst_flat)
    s1 = s1.reshape(T, NPAD, H)

    y2 = _mm2(s1, y1, deg3, b1.reshape(1, H), W2)
    s2 = _make_agg_kernel()(y2.reshape(T * N, H), src_flat, dst_flat)
    s2 = s2.reshape(T, NPAD, H)

    egof = jnp.transpose(ego_mask, (1, 0, 2)).reshape(T, N, 1).astype(
        jnp.float32)

    out = _final(s2, y2, deg3, egof, b2.reshape(1, H),
                 W_ih.reshape(4, H, H), W_hh.reshape(4, H, H),
                 b_ih.reshape(4, H), b_hh.reshape(4, H),
                 W_fc, b_fc.reshape(1, D_OUT))
    return out.reshape(B, 400, D_OUT)
